# baseline (speedup 1.0000x reference)
import jax
import jax.numpy as jnp
from jax import lax
from jax.experimental import pallas as pl
from jax.experimental.pallas import tpu as pltpu
from jax.experimental.pallas import tpu_sc as plsc

B = 1024
SEQ = 200
MLEN = 20
EMB = 64
WPE = 16
OUTD = EMB + WPE
PAD = 128
NC, NS, L = 2, 16, 16
NW = NC * NS
BPW = B // NW
RPW = BPW * SEQ
MPW = BPW * MLEN
SEQ_A = 128
SEQ_B = SEQ - SEQ_A


def _body(words_hbm, ments_hbm, mlen_hbm, pos_hbm, w2_hbm, wpos_hbm,
          out_sent, out_ment, out_pool,
          widx, pidx, midx, mlen_v, comb0, comb1, combo, mrow0, mrow1, mbo,
          wposv, pool_v, w0v, gsem, wsem, msem):
    wid = lax.axis_index("s") * NC + lax.axis_index("c")
    b0 = wid * BPW

    pltpu.sync_copy(words_hbm.at[wid], widx)
    pltpu.sync_copy(pos_hbm.at[wid], pidx)
    pltpu.sync_copy(ments_hbm.at[wid], midx)
    pltpu.sync_copy(mlen_hbm.at[pl.ds(b0, BPW)], mlen_v)
    pltpu.sync_copy(wpos_hbm, wposv)
    pltpu.sync_copy(w2_hbm.at[pl.ds(0, 8)], w0v)

    combs = (comb0, comb1)
    mrows = (mrow0, mrow1)

    def pool_init(bb, _):
        ml = plsc.load_gather(mlen_v, [jnp.full((L,), bb, jnp.int32)])
        rem = (MLEN - jnp.maximum(ml - 2, 0)).astype(jnp.float32)
        for c in range(EMB // L):
            pool_v[bb, pl.ds(c * L, L)] = rem * w0v[0, pl.ds(c * L, L)]
        return _

    lax.fori_loop(0, BPW, pool_init, 0)

    def fire_m(p, buf):
        return pltpu.async_copy(
            w2_hbm.at[midx.at[pl.ds(p * 2 * MLEN, 2 * MLEN)]], buf, msem)

    NPAIR = BPW // 2

    def process_pair(p, buf):
        for half in range(2):
            bb_l = 2 * p + half
            ml = plsc.load_gather(mlen_v, [jnp.full((L,), bb_l, jnp.int32)])

            def jloop(j, _):
                row = half * MLEN + j
                jv = jnp.full((L,), j, jnp.int32)
                w = jnp.where((jv >= 1) & (jv < ml - 1), 1.0, 0.0)
                for c in range(EMB // L):
                    v = buf[row, pl.ds(c * L, L)]
                    mbo[j, pl.ds(c * L, L)] = v
                    plsc.addupdate(pool_v.at[bb_l, pl.ds(c * L, L)], v * w)
                return _

            lax.fori_loop(0, MLEN, jloop, 0)
            pltpu.sync_copy(mbo, out_ment.at[b0 + bb_l])

    def ment_iter(p, _):
        even = (p % 2) == 0

        def run(cur, oth):
            pltpu.make_async_copy(w2_hbm.at[pl.ds(0, 2 * MLEN)], cur,
                                  msem).wait()

            @pl.when(p < NPAIR - 1)
            def _fire_next():
                fire_m(p + 1, oth)

            process_pair(p, cur)

        @pl.when(even)
        def _even():
            run(mrow0, mrow1)

        @pl.when(jnp.logical_not(even))
        def _odd():
            run(mrow1, mrow0)

        return _

    fire_m(0, mrows[0])
    lax.fori_loop(0, NPAIR, ment_iter, 0)

    def pool_fin(bb, _):
        ml = plsc.load_gather(mlen_v, [jnp.full((L,), bb, jnp.int32)])
        denom = jnp.maximum(ml - 2, 1).astype(jnp.float32)
        for c in range(EMB // L):
            pool_v[bb, pl.ds(c * L, L)] = pool_v[bb, pl.ds(c * L, L)] / denom
        return _

    lax.fori_loop(0, BPW, pool_fin, 0)
    pltpu.sync_copy(pool_v, out_pool.at[pl.ds(b0, BPW)])

    iota = lax.iota(jnp.int32, L)

    def fire_s(bb, buf):
        base = bb * SEQ
        pltpu.async_copy(w2_hbm.at[widx.at[pl.ds(base, SEQ_A)]],
                         buf.at[pl.ds(0, SEQ_A)], gsem)
        pltpu.async_copy(w2_hbm.at[widx.at[pl.ds(base + SEQ_A, SEQ_B)]],
                         buf.at[pl.ds(SEQ_A, SEQ_B)], gsem)

    def wait_s(buf):
        pltpu.make_async_copy(w2_hbm.at[pl.ds(0, SEQ)], buf, gsem).wait()

    def fill_rows(bb, buf):
        def fill4(k, _):
            r = k * 4
            for u in range(4):
                for c in range(EMB // L):
                    combo[r + u, pl.ds(c * L, L)] = buf[r + u, pl.ds(c * L, L)]
                p = plsc.load_gather(
                    pidx, [jnp.full((L,), bb * SEQ + r + u, jnp.int32)])
                pv = plsc.load_gather(wposv, [(p >> 3),
                                              (p & 7) * WPE + iota])
                combo[r + u, pl.ds(EMB, WPE)] = pv
            return _

        lax.fori_loop(0, SEQ // 4, fill4, 0)

    def sent_iter(bb, _):
        even = (bb % 2) == 0

        def run(cur, oth):
            wait_s(cur)

            @pl.when(bb < BPW - 1)
            def _fire_next():
                fire_s(bb + 1, oth)

            @pl.when(bb > 0)
            def _wait_prev_write():
                pltpu.make_async_copy(combo, out_sent.at[b0], wsem).wait()

            fill_rows(bb, cur)
            pltpu.async_copy(combo, out_sent.at[b0 + bb], wsem)

        @pl.when(even)
        def _even():
            run(comb0, comb1)

        @pl.when(jnp.logical_not(even))
        def _odd():
            run(comb1, comb0)

        return _

    fire_s(0, comb0)
    lax.fori_loop(0, BPW, sent_iter, 0)
    pltpu.make_async_copy(combo, out_sent.at[b0], wsem).wait()


def kernel(input_words, input_mentions, input_mentionlen, input_positions, W_embed, W_pos):
    words2 = input_words.reshape(NW, RPW).astype(jnp.int32)
    pos2 = input_positions.reshape(NW, RPW).astype(jnp.int32)
    ments2 = input_mentions.reshape(NW, MPW).astype(jnp.int32)
    mlen = input_mentionlen.astype(jnp.int32)
    w2 = jnp.pad(W_embed, ((0, 0), (0, PAD - EMB)))
    wposf = jnp.pad(W_pos.reshape(-1), (0, 64 * PAD - 500 * WPE))
    wposf = wposf.reshape(64, PAD)
    mesh = plsc.VectorSubcoreMesh(core_axis_name="c", subcore_axis_name="s")
    out_sent, out_ment, out_pool = pl.kernel(
        _body,
        out_type=[
            jax.ShapeDtypeStruct((B, SEQ, OUTD), jnp.float32),
            jax.ShapeDtypeStruct((B, MLEN, EMB), jnp.float32),
            jax.ShapeDtypeStruct((B, EMB), jnp.float32),
        ],
        mesh=mesh,
        compiler_params=pltpu.CompilerParams(needs_layout_passes=False),
        scratch_types=[
            pltpu.VMEM((RPW,), jnp.int32),
            pltpu.VMEM((RPW,), jnp.int32),
            pltpu.VMEM((MPW,), jnp.int32),
            pltpu.VMEM((BPW,), jnp.int32),
            pltpu.VMEM((SEQ, PAD), jnp.float32),
            pltpu.VMEM((SEQ, PAD), jnp.float32),
            pltpu.VMEM((SEQ, OUTD), jnp.float32),
            pltpu.VMEM((2 * MLEN, PAD), jnp.float32),
            pltpu.VMEM((2 * MLEN, PAD), jnp.float32),
            pltpu.VMEM((MLEN, EMB), jnp.float32),
            pltpu.VMEM((64, PAD), jnp.float32),
            pltpu.VMEM((BPW, EMB), jnp.float32),
            pltpu.VMEM((8, PAD), jnp.float32),
            pltpu.SemaphoreType.DMA,
            pltpu.SemaphoreType.DMA,
            pltpu.SemaphoreType.DMA,
        ],
    )(words2, ments2, mlen, pos2, w2, wposf)
    return (out_sent, out_ment, out_pool)


# speedup vs baseline: 1.1126x; 1.1126x over previous
import jax
import jax.numpy as jnp
from jax import lax
from jax.experimental import pallas as pl
from jax.experimental.pallas import tpu as pltpu
from jax.experimental.pallas import tpu_sc as plsc

B = 1024
SEQ = 200
MLEN = 20
EMB = 64
WPE = 16
OUTD = EMB + WPE
PAD = 128
NC, NS, L = 2, 16, 16
NW = NC * NS
BPW = B // NW
RPW = BPW * SEQ
MPW = BPW * MLEN
SEQ_A = 128
SEQ_B = SEQ - SEQ_A


def _body(words_hbm, ments_hbm, mlen_hbm, pos_hbm, w2_hbm, wpos_hbm,
          out_sent, out_ment, out_pool,
          widx, pidx, midx, mlen_v, comb0, comb1, combo, mrow0, mrow1, mbo,
          wposv, pool_v, w0v, gsem, wsem, msem):
    wid = lax.axis_index("s") * NC + lax.axis_index("c")
    b0 = wid * BPW

    pltpu.sync_copy(words_hbm.at[wid], widx)
    pltpu.sync_copy(pos_hbm.at[wid], pidx)
    pltpu.sync_copy(ments_hbm.at[wid], midx)
    pltpu.sync_copy(mlen_hbm.at[pl.ds(b0, BPW)], mlen_v)
    pltpu.sync_copy(wpos_hbm, wposv)
    pltpu.sync_copy(w2_hbm.at[pl.ds(0, 8)], w0v)

    combs = (comb0, comb1)
    mrows = (mrow0, mrow1)

    def pool_init(bb, _):
        ml = plsc.load_gather(mlen_v, [jnp.full((L,), bb, jnp.int32)])
        rem = (MLEN - jnp.maximum(ml - 2, 0)).astype(jnp.float32)
        for c in range(EMB // L):
            pool_v[bb, pl.ds(c * L, L)] = rem * w0v[0, pl.ds(c * L, L)]
        return _

    lax.fori_loop(0, BPW, pool_init, 0)

    def fire_m(p, buf):
        return pltpu.async_copy(
            w2_hbm.at[midx.at[pl.ds(p * 2 * MLEN, 2 * MLEN)]], buf, msem)

    NPAIR = BPW // 2

    def process_pair(p, buf):
        for half in range(2):
            bb_l = 2 * p + half
            ml = plsc.load_gather(mlen_v, [jnp.full((L,), bb_l, jnp.int32)])

            def jloop(j, _):
                row = half * MLEN + j
                jv = jnp.full((L,), j, jnp.int32)
                w = jnp.where((jv >= 1) & (jv < ml - 1), 1.0, 0.0)
                for c in range(EMB // L):
                    v = buf[row, pl.ds(c * L, L)]
                    mbo[j, pl.ds(c * L, L)] = v
                    plsc.addupdate(pool_v.at[bb_l, pl.ds(c * L, L)], v * w)
                return _

            lax.fori_loop(0, MLEN, jloop, 0)
            pltpu.sync_copy(mbo, out_ment.at[b0 + bb_l])

    def ment_iter(p, _):
        even = (p % 2) == 0

        def run(cur, oth):
            pltpu.make_async_copy(w2_hbm.at[pl.ds(0, 2 * MLEN)], cur,
                                  msem).wait()

            @pl.when(p < NPAIR - 1)
            def _fire_next():
                fire_m(p + 1, oth)

            process_pair(p, cur)

        @pl.when(even)
        def _even():
            run(mrow0, mrow1)

        @pl.when(jnp.logical_not(even))
        def _odd():
            run(mrow1, mrow0)

        return _

    fire_m(0, mrows[0])
    lax.fori_loop(0, NPAIR, ment_iter, 0)

    def pool_fin(bb, _):
        ml = plsc.load_gather(mlen_v, [jnp.full((L,), bb, jnp.int32)])
        denom = jnp.maximum(ml - 2, 1).astype(jnp.float32)
        for c in range(EMB // L):
            pool_v[bb, pl.ds(c * L, L)] = pool_v[bb, pl.ds(c * L, L)] / denom
        return _

    lax.fori_loop(0, BPW, pool_fin, 0)
    pltpu.sync_copy(pool_v, out_pool.at[pl.ds(b0, BPW)])

    iota = lax.iota(jnp.int32, L)

    def fire_s(bb, buf):
        base = bb * SEQ
        pltpu.async_copy(w2_hbm.at[widx.at[pl.ds(base, SEQ_A)]],
                         buf.at[pl.ds(0, SEQ_A)], gsem)
        pltpu.async_copy(w2_hbm.at[widx.at[pl.ds(base + SEQ_A, SEQ_B)]],
                         buf.at[pl.ds(SEQ_A, SEQ_B)], gsem)

    def wait_s(buf):
        pltpu.make_async_copy(w2_hbm.at[pl.ds(0, SEQ)], buf, gsem).wait()

    def fill_rows(bb, buf):
        def fill2(k, _):
            r = k * 2
            for u in range(2):
                for c in range(EMB // L):
                    combo[r + u, pl.ds(c * L, L)] = buf[r + u, pl.ds(c * L, L)]
                p = plsc.load_gather(
                    pidx, [jnp.full((L,), bb * SEQ + r + u, jnp.int32)])
                pv = plsc.load_gather(wposv, [(p >> 3),
                                              (p & 7) * WPE + iota])
                combo[r + u, pl.ds(EMB, WPE)] = pv
            return _

        lax.fori_loop(0, SEQ // 2, fill2, 0)

    def sent_iter(bb, _):
        even = (bb % 2) == 0

        def run(cur, oth):
            wait_s(cur)

            @pl.when(bb > 0)
            def _wait_prev_write():
                pltpu.make_async_copy(combo, out_sent.at[b0], wsem).wait()

            fill_rows(bb, cur)
            pltpu.async_copy(combo, out_sent.at[b0 + bb], wsem)

            @pl.when(bb < BPW - 1)
            def _fire_next():
                fire_s(bb + 1, oth)

        @pl.when(even)
        def _even():
            run(comb0, comb1)

        @pl.when(jnp.logical_not(even))
        def _odd():
            run(comb1, comb0)

        return _

    fire_s(0, comb0)
    lax.fori_loop(0, BPW, sent_iter, 0)
    pltpu.make_async_copy(combo, out_sent.at[b0], wsem).wait()


def kernel(input_words, input_mentions, input_mentionlen, input_positions, W_embed, W_pos):
    words2 = input_words.reshape(NW, RPW).astype(jnp.int32)
    pos2 = input_positions.reshape(NW, RPW).astype(jnp.int32)
    ments2 = input_mentions.reshape(NW, MPW).astype(jnp.int32)
    mlen = input_mentionlen.astype(jnp.int32)
    w2 = jnp.pad(W_embed, ((0, 0), (0, PAD - EMB)))
    wposf = jnp.pad(W_pos.reshape(-1), (0, 64 * PAD - 500 * WPE))
    wposf = wposf.reshape(64, PAD)
    mesh = plsc.VectorSubcoreMesh(core_axis_name="c", subcore_axis_name="s")
    out_sent, out_ment, out_pool = pl.kernel(
        _body,
        out_type=[
            jax.ShapeDtypeStruct((B, SEQ, OUTD), jnp.float32),
            jax.ShapeDtypeStruct((B, MLEN, EMB), jnp.float32),
            jax.ShapeDtypeStruct((B, EMB), jnp.float32),
        ],
        mesh=mesh,
        compiler_params=pltpu.CompilerParams(needs_layout_passes=False),
        scratch_types=[
            pltpu.VMEM((RPW,), jnp.int32),
            pltpu.VMEM((RPW,), jnp.int32),
            pltpu.VMEM((MPW,), jnp.int32),
            pltpu.VMEM((BPW,), jnp.int32),
            pltpu.VMEM((SEQ, PAD), jnp.float32),
            pltpu.VMEM((SEQ, PAD), jnp.float32),
            pltpu.VMEM((SEQ, OUTD), jnp.float32),
            pltpu.VMEM((2 * MLEN, PAD), jnp.float32),
            pltpu.VMEM((2 * MLEN, PAD), jnp.float32),
            pltpu.VMEM((MLEN, EMB), jnp.float32),
            pltpu.VMEM((64, PAD), jnp.float32),
            pltpu.VMEM((BPW, EMB), jnp.float32),
            pltpu.VMEM((8, PAD), jnp.float32),
            pltpu.SemaphoreType.DMA,
            pltpu.SemaphoreType.DMA,
            pltpu.SemaphoreType.DMA,
        ],
    )(words2, ments2, mlen, pos2, w2, wposf)
    return (out_sent, out_ment, out_pool)


# speedup vs baseline: 1.2735x; 1.1446x over previous
import jax
import jax.numpy as jnp
from jax import lax
from jax.experimental import pallas as pl
from jax.experimental.pallas import tpu as pltpu
from jax.experimental.pallas import tpu_sc as plsc

B = 1024
SEQ = 200
MLEN = 20
EMB = 64
WPE = 16
OUTD = EMB + WPE
PAD = 128
NC, NS, L = 2, 16, 16
NW = NC * NS
BPW = B // NW
RPW = BPW * SEQ
MPW = BPW * MLEN
SEQ_A = 128
SEQ_B = SEQ - SEQ_A


def _body(words_hbm, ments_hbm, mlen_hbm, pos_hbm, w2_hbm, wpos_hbm,
          out_sent, out_ment, out_pool,
          widx, pidx, midx, mlen_v, comb0, comb1, combo, mrow0, mrow1, mbo,
          wposv, pool_v, w0v, gsem, wsem, msem):
    wid = lax.axis_index("s") * NC + lax.axis_index("c")
    b0 = wid * BPW

    pltpu.sync_copy(words_hbm.at[wid], widx)
    pltpu.sync_copy(pos_hbm.at[wid], pidx)
    pltpu.sync_copy(ments_hbm.at[wid], midx)
    pltpu.sync_copy(mlen_hbm.at[pl.ds(b0, BPW)], mlen_v)
    pltpu.sync_copy(wpos_hbm, wposv)
    pltpu.sync_copy(w2_hbm.at[pl.ds(0, 8)], w0v)

    combs = (comb0, comb1)
    mrows = (mrow0, mrow1)

    def pool_init(bb, _):
        ml = plsc.load_gather(mlen_v, [jnp.full((L,), bb, jnp.int32)])
        rem = (MLEN - jnp.maximum(ml - 2, 0)).astype(jnp.float32)
        for c in range(EMB // L):
            pool_v[bb, pl.ds(c * L, L)] = rem * w0v[0, pl.ds(c * L, L)]
        return _

    lax.fori_loop(0, BPW, pool_init, 0)

    def fire_m(p, buf):
        return pltpu.async_copy(
            w2_hbm.at[midx.at[pl.ds(p * 2 * MLEN, 2 * MLEN)]], buf, msem)

    NPAIR = BPW // 2

    def process_pair(p, buf):
        for half in range(2):
            bb_l = 2 * p + half
            ml = plsc.load_gather(mlen_v, [jnp.full((L,), bb_l, jnp.int32)])

            def jloop(j, _):
                row = half * MLEN + j
                jv = jnp.full((L,), j, jnp.int32)
                w = jnp.where((jv >= 1) & (jv < ml - 1), 1.0, 0.0)
                for c in range(EMB // L):
                    v = buf[row, pl.ds(c * L, L)]
                    mbo[j, pl.ds(c * L, L)] = v
                    plsc.addupdate(pool_v.at[bb_l, pl.ds(c * L, L)], v * w)
                return _

            lax.fori_loop(0, MLEN, jloop, 0)
            pltpu.sync_copy(mbo, out_ment.at[b0 + bb_l])

    def ment_iter(p, _):
        even = (p % 2) == 0

        def run(cur, oth):
            pltpu.make_async_copy(w2_hbm.at[pl.ds(0, 2 * MLEN)], cur,
                                  msem).wait()

            @pl.when(p < NPAIR - 1)
            def _fire_next():
                fire_m(p + 1, oth)

            process_pair(p, cur)

        @pl.when(even)
        def _even():
            run(mrow0, mrow1)

        @pl.when(jnp.logical_not(even))
        def _odd():
            run(mrow1, mrow0)

        return _

    fire_m(0, mrows[0])
    lax.fori_loop(0, NPAIR, ment_iter, 0)

    def pool_fin(bb, _):
        ml = plsc.load_gather(mlen_v, [jnp.full((L,), bb, jnp.int32)])
        denom = jnp.maximum(ml - 2, 1).astype(jnp.float32)
        for c in range(EMB // L):
            pool_v[bb, pl.ds(c * L, L)] = pool_v[bb, pl.ds(c * L, L)] / denom
        return _

    lax.fori_loop(0, BPW, pool_fin, 0)
    pltpu.sync_copy(pool_v, out_pool.at[pl.ds(b0, BPW)])

    iota = lax.iota(jnp.int32, L)

    def fire_s(bb, buf):
        base = bb * SEQ
        pltpu.async_copy(w2_hbm.at[widx.at[pl.ds(base, SEQ_A)]],
                         buf.at[pl.ds(0, SEQ_A)], gsem)
        pltpu.async_copy(w2_hbm.at[widx.at[pl.ds(base + SEQ_A, SEQ_B)]],
                         buf.at[pl.ds(SEQ_A, SEQ_B)], gsem)

    def wait_s(buf):
        pltpu.make_async_copy(w2_hbm.at[pl.ds(0, SEQ)], buf, gsem).wait()

    def fill_rows(bb, buf):
        def fill2(k, _):
            r = k * 2
            for u in range(2):
                for c in range(EMB // L):
                    combo[r + u, pl.ds(c * L, L)] = buf[r + u, pl.ds(c * L, L)]
                p = plsc.load_gather(
                    pidx, [jnp.full((L,), bb * SEQ + r + u, jnp.int32)])
                pv = plsc.load_gather(wposv, [(p >> 3),
                                              (p & 7) * WPE + iota])
                combo[r + u, pl.ds(EMB, WPE)] = pv
            return _

        lax.fori_loop(0, SEQ // 2, fill2, 0)

    def sent_iter(bb, _):
        even = (bb % 2) == 0

        def run(cur, oth):
            wait_s(cur)

            @pl.when(bb < BPW - 1)
            def _fire_next():
                fire_s(bb + 1, oth)

            @pl.when(bb > 0)
            def _wait_prev_write():
                pltpu.make_async_copy(combo, out_sent.at[b0], wsem).wait()

            fill_rows(bb, cur)
            pltpu.async_copy(combo, out_sent.at[b0 + bb], wsem)

        @pl.when(even)
        def _even():
            run(comb0, comb1)

        @pl.when(jnp.logical_not(even))
        def _odd():
            run(comb1, comb0)

        return _

    fire_s(0, comb0)
    lax.fori_loop(0, BPW, sent_iter, 0)
    pltpu.make_async_copy(combo, out_sent.at[b0], wsem).wait()


def kernel(input_words, input_mentions, input_mentionlen, input_positions, W_embed, W_pos):
    words2 = input_words.reshape(NW, RPW).astype(jnp.int32)
    pos2 = input_positions.reshape(NW, RPW).astype(jnp.int32)
    ments2 = input_mentions.reshape(NW, MPW).astype(jnp.int32)
    mlen = input_mentionlen.astype(jnp.int32)
    w2 = jnp.pad(W_embed, ((0, 0), (0, PAD - EMB)))
    wposf = jnp.pad(W_pos.reshape(-1), (0, 64 * PAD - 500 * WPE))
    wposf = wposf.reshape(64, PAD)
    mesh = plsc.VectorSubcoreMesh(core_axis_name="c", subcore_axis_name="s")
    out_sent, out_ment, out_pool = pl.kernel(
        _body,
        out_type=[
            jax.ShapeDtypeStruct((B, SEQ, OUTD), jnp.float32),
            jax.ShapeDtypeStruct((B, MLEN, EMB), jnp.float32),
            jax.ShapeDtypeStruct((B, EMB), jnp.float32),
        ],
        mesh=mesh,
        compiler_params=pltpu.CompilerParams(needs_layout_passes=False),
        scratch_types=[
            pltpu.VMEM((RPW,), jnp.int32),
            pltpu.VMEM((RPW,), jnp.int32),
            pltpu.VMEM((MPW,), jnp.int32),
            pltpu.VMEM((BPW,), jnp.int32),
            pltpu.VMEM((SEQ, PAD), jnp.float32),
            pltpu.VMEM((SEQ, PAD), jnp.float32),
            pltpu.VMEM((SEQ, OUTD), jnp.float32),
            pltpu.VMEM((2 * MLEN, PAD), jnp.float32),
            pltpu.VMEM((2 * MLEN, PAD), jnp.float32),
            pltpu.VMEM((MLEN, EMB), jnp.float32),
            pltpu.VMEM((64, PAD), jnp.float32),
            pltpu.VMEM((BPW, EMB), jnp.float32),
            pltpu.VMEM((8, PAD), jnp.float32),
            pltpu.SemaphoreType.DMA,
            pltpu.SemaphoreType.DMA,
            pltpu.SemaphoreType.DMA,
        ],
    )(words2, ments2, mlen, pos2, w2, wposf)
    return (out_sent, out_ment, out_pool)
